# Initial kernel scaffold; baseline (speedup 1.0000x reference)
import jax
import jax.numpy as jnp
from jax.experimental import pallas as pl


def kernel(hidden_states, gate_w, bias, w1, w3, w2, sh_wg, sh_wu, sh_wd):
    raise NotImplementedError("write your pallas kernel here")


# speedup vs baseline: 1.1559x; 1.1559x over previous
import functools

import jax
import jax.numpy as jnp
from jax import lax
from jax.experimental import pallas as pl
from jax.experimental.pallas import tpu as pltpu

E = 16
TOPK = 2
H = 2048
FF = 1024
SHARED_FF = 1024
T = 2048

_INTERP = False


def _bf16_dot(a, b, dims):
    return lax.dot_general(a.astype(jnp.bfloat16), b.astype(jnp.bfloat16),
                           dims, preferred_element_type=jnp.float32)


def _gate_shared_body(x_ref, gate_w_ref, bias_ref, wg_ref, wu_ref, wd_ref,
                      shared_ref, wtok_ref):
    x = x_ref[...]
    g = _bf16_dot(x, wg_ref[...], (((1,), (1,)), ((), ())))
    u = _bf16_dot(x, wu_ref[...], (((1,), (1,)), ((), ())))
    h = (g * jax.nn.sigmoid(g)) * u
    shared_ref[...] = _bf16_dot(h, wd_ref[...], (((1,), (1,)), ((), ())))
    logits = lax.dot_general(x, gate_w_ref[...], (((1,), (1,)), ((), ())),
                             preferred_element_type=jnp.float32)
    s = jax.nn.sigmoid(logits)
    sc = s + bias_ref[...]
    iota = lax.broadcasted_iota(jnp.int32, sc.shape, 1)
    m1 = jnp.max(sc, axis=1, keepdims=True)
    i1 = jnp.min(jnp.where(sc == m1, iota, E), axis=1, keepdims=True)
    sc2 = jnp.where(iota == i1, -jnp.inf, sc)
    m2 = jnp.max(sc2, axis=1, keepdims=True)
    i2 = jnp.min(jnp.where(sc2 == m2, iota, E), axis=1, keepdims=True)
    w1v = jnp.sum(jnp.where(iota == i1, s, 0.0), axis=1, keepdims=True)
    w2v = jnp.sum(jnp.where(iota == i2, s, 0.0), axis=1, keepdims=True)
    denom = w1v + w2v
    wtok_ref[...] = (jnp.where(iota == i1, w1v, 0.0)
                     + jnp.where(iota == i2, w2v, 0.0)) / denom


def _experts_body(x_ref, wtok_ref, w1_ref, w3_ref, w2_ref, out_ref):
    e = pl.program_id(0)
    f = pl.program_id(1)
    first = jnp.logical_and(e == 0, f == 0)
    BT = 256
    for tb in range(T // BT):
        xs = x_ref[tb * BT:(tb + 1) * BT, :]
        g = _bf16_dot(xs, w1_ref[0], (((1,), (1,)), ((), ())))
        u = _bf16_dot(xs, w3_ref[0], (((1,), (1,)), ((), ())))
        h = (g * jax.nn.sigmoid(g)) * u
        y = _bf16_dot(h, w2_ref[0], (((1,), (1,)), ((), ())))
        wt = wtok_ref[tb * BT:(tb + 1) * BT, :]
        lane = lax.broadcasted_iota(jnp.int32, wt.shape, 1)
        w_col = jnp.sum(jnp.where(lane == e, wt, 0.0), axis=1)
        y = y * w_col[:, None]

        @pl.when(first)
        def _init():
            out_ref[tb * BT:(tb + 1) * BT, :] = y

        @pl.when(jnp.logical_not(first))
        def _acc():
            out_ref[tb * BT:(tb + 1) * BT, :] += y


def _add_body(a_ref, b_ref, o_ref):
    o_ref[...] = a_ref[...] + b_ref[...]


def kernel(hidden_states, gate_w, bias, w1, w3, w2, sh_wg, sh_wu, sh_wd):
    x = hidden_states
    BT = 256
    shared, wtok = pl.pallas_call(
        _gate_shared_body,
        grid=(T // BT,),
        in_specs=[
            pl.BlockSpec((BT, H), lambda t: (t, 0)),
            pl.BlockSpec((E, H), lambda t: (0, 0)),
            pl.BlockSpec((1, E), lambda t: (0, 0)),
            pl.BlockSpec((SHARED_FF, H), lambda t: (0, 0)),
            pl.BlockSpec((SHARED_FF, H), lambda t: (0, 0)),
            pl.BlockSpec((H, SHARED_FF), lambda t: (0, 0)),
        ],
        out_specs=[
            pl.BlockSpec((BT, H), lambda t: (t, 0)),
            pl.BlockSpec((BT, E), lambda t: (t, 0)),
        ],
        out_shape=[
            jax.ShapeDtypeStruct((T, H), jnp.float32),
            jax.ShapeDtypeStruct((T, E), jnp.float32),
        ],
        interpret=_INTERP,
    )(x, gate_w, bias, sh_wg, sh_wu, sh_wd)

    BF = 256
    eout = pl.pallas_call(
        _experts_body,
        grid=(E, FF // BF),
        in_specs=[
            pl.BlockSpec((T, H), lambda e, f: (0, 0)),
            pl.BlockSpec((T, E), lambda e, f: (0, 0)),
            pl.BlockSpec((1, BF, H), lambda e, f: (e, f, 0)),
            pl.BlockSpec((1, BF, H), lambda e, f: (e, f, 0)),
            pl.BlockSpec((1, H, BF), lambda e, f: (e, 0, f)),
        ],
        out_specs=pl.BlockSpec((T, H), lambda e, f: (0, 0)),
        out_shape=jax.ShapeDtypeStruct((T, H), jnp.float32),
        interpret=_INTERP,
    )(x, wtok, w1, w3, w2)
    out = pl.pallas_call(
        _add_body,
        grid=(T // BT,),
        in_specs=[
            pl.BlockSpec((BT, H), lambda t: (t, 0)),
            pl.BlockSpec((BT, H), lambda t: (t, 0)),
        ],
        out_specs=pl.BlockSpec((BT, H), lambda t: (t, 0)),
        out_shape=jax.ShapeDtypeStruct((T, H), jnp.float32),
        interpret=_INTERP,
    )(eout, shared)
    return out


# speedup vs baseline: 1.4969x; 1.2950x over previous
import functools

import jax
import jax.numpy as jnp
from jax import lax
from jax.experimental import pallas as pl
from jax.experimental.pallas import tpu as pltpu
from jax.experimental.pallas import tpu_sc as plsc

E = 16
TOPK = 2
H = 2048
FF = 1024
SHARED_FF = 1024
T = 2048

NPAIR = T * TOPK
RT = 128
NT = NPAIR // RT + E
NP = NT * RT
NW = 32

_INTERP = False


def _bf16_dot(a, b, dims):
    return lax.dot_general(a.astype(jnp.bfloat16), b.astype(jnp.bfloat16),
                           dims, preferred_element_type=jnp.float32)


def _k1_body(x_ref, gate_w_ref, bias_ref, wg_ref, wu_ref, wd_ref,
             shared_ref, wpair_ref, idx_ref, rank_ref, cnt_ref, carry_ref):
    t = pl.program_id(0)
    x = x_ref[...]
    g = _bf16_dot(x, wg_ref[...], (((1,), (1,)), ((), ())))
    u = _bf16_dot(x, wu_ref[...], (((1,), (1,)), ((), ())))
    h = (g * jax.nn.sigmoid(g)) * u
    shared_ref[...] = _bf16_dot(h, wd_ref[...], (((1,), (1,)), ((), ())))
    logits = lax.dot_general(x, gate_w_ref[...], (((1,), (1,)), ((), ())),
                             preferred_element_type=jnp.float32)
    s = jax.nn.sigmoid(logits)
    sc = s + bias_ref[...]
    iota = lax.broadcasted_iota(jnp.int32, sc.shape, 1)
    m1 = jnp.max(sc, axis=1, keepdims=True)
    i1 = jnp.min(jnp.where(sc == m1, iota, E), axis=1, keepdims=True)
    sc2 = jnp.where(iota == i1, -jnp.inf, sc)
    m2 = jnp.max(sc2, axis=1, keepdims=True)
    i2 = jnp.min(jnp.where(sc2 == m2, iota, E), axis=1, keepdims=True)
    w1v = jnp.sum(jnp.where(iota == i1, s, 0.0), axis=1, keepdims=True)
    w2v = jnp.sum(jnp.where(iota == i2, s, 0.0), axis=1, keepdims=True)
    denom = w1v + w2v
    wpair_ref[...] = jnp.concatenate([w1v / denom, w2v / denom], axis=1)
    idx_ref[...] = jnp.concatenate([i1, i2], axis=1)

    @pl.when(t == 0)
    def _init_carry():
        carry_ref[...] = jnp.zeros_like(carry_ref)

    oh1 = (iota == i1).astype(jnp.float32)
    oh2 = (iota == i2).astype(jnp.float32)
    ohs = oh1 + oh2
    bt = oh1.shape[0]
    r_io = lax.broadcasted_iota(jnp.int32, (bt, bt), 0)
    c_io = lax.broadcasted_iota(jnp.int32, (bt, bt), 1)
    tril = (r_io > c_io).astype(jnp.float32)
    cum = lax.dot_general(tril, ohs, (((1,), (0,)), ((), ())),
                          preferred_element_type=jnp.float32)
    base = cum + carry_ref[...]
    r1 = jnp.sum(oh1 * base, axis=1, keepdims=True)
    r2 = jnp.sum(oh2 * base, axis=1, keepdims=True)
    rank_ref[...] = jnp.concatenate([r1, r2], axis=1).astype(jnp.int32)
    carry_ref[...] += jnp.sum(ohs, axis=0, keepdims=True)
    cnt_ref[...] = jnp.broadcast_to(carry_ref[...], cnt_ref.shape).astype(jnp.int32)


def _k1(x, gate_w, bias, sh_wg, sh_wu, sh_wd):
    BT = 256
    return pl.pallas_call(
        _k1_body,
        grid=(T // BT,),
        in_specs=[
            pl.BlockSpec((BT, H), lambda t: (t, 0)),
            pl.BlockSpec((E, H), lambda t: (0, 0)),
            pl.BlockSpec((1, E), lambda t: (0, 0)),
            pl.BlockSpec((SHARED_FF, H), lambda t: (0, 0)),
            pl.BlockSpec((SHARED_FF, H), lambda t: (0, 0)),
            pl.BlockSpec((H, SHARED_FF), lambda t: (0, 0)),
        ],
        out_specs=[
            pl.BlockSpec((BT, H), lambda t: (t, 0)),
            pl.BlockSpec((BT, TOPK), lambda t: (t, 0)),
            pl.BlockSpec((BT, TOPK), lambda t: (t, 0)),
            pl.BlockSpec((BT, TOPK), lambda t: (t, 0)),
            pl.BlockSpec((8, E), lambda t: (0, 0)),
        ],
        out_shape=[
            jax.ShapeDtypeStruct((T, H), jnp.float32),
            jax.ShapeDtypeStruct((T, TOPK), jnp.float32),
            jax.ShapeDtypeStruct((T, TOPK), jnp.int32),
            jax.ShapeDtypeStruct((T, TOPK), jnp.int32),
            jax.ShapeDtypeStruct((8, E), jnp.int32),
        ],
        scratch_shapes=[pltpu.VMEM((1, E), jnp.float32)],
        interpret=_INTERP,
    )(x, gate_w, bias, sh_wg, sh_wu, sh_wd)


def _take(v, idxv):
    return lax.gather(
        v, idxv[:, None],
        lax.GatherDimensionNumbers(offset_dims=(), collapsed_slice_dims=(0,),
                                   start_index_map=(0,)),
        (1,), mode=lax.GatherScatterMode.PROMISE_IN_BOUNDS)


def _bcast_lane(v, j):
    return _take(v, jnp.full((16,), j, jnp.int32))


def _k1b_body(cnt_ref, idx_ref, rank_ref, pos_ref, te_ref, act_ref):
    cnt = cnt_ref[0:1, :]
    padc = ((cnt + (RT - 1)) >> 7) << 7
    padc_f = padc.astype(jnp.float32)
    r_io = lax.broadcasted_iota(jnp.int32, (E, E), 0)
    c_io = lax.broadcasted_iota(jnp.int32, (E, E), 1)
    supper = (r_io < c_io).astype(jnp.float32)
    po = lax.dot_general(padc_f, supper, (((1,), (0,)), ((), ())),
                         preferred_element_type=jnp.float32)
    ends = po + padc_f
    total = jnp.sum(padc_f, axis=1, keepdims=True)
    iota = lax.broadcasted_iota(jnp.int32, (T, E), 1)
    i1 = idx_ref[:, 0:1]
    i2 = idx_ref[:, 1:2]
    p1 = jnp.sum(jnp.where(iota == i1, po, 0.0), axis=1, keepdims=True)
    p2 = jnp.sum(jnp.where(iota == i2, po, 0.0), axis=1, keepdims=True)
    pos = jnp.concatenate([p1, p2], axis=1).astype(jnp.int32) + rank_ref[...]
    pos_ref[...] = pos
    lane64 = lax.broadcasted_iota(jnp.int32, (1, 64), 1)
    ts = (lane64 * RT).astype(jnp.float32)
    te = jnp.zeros((1, 64), jnp.float32)
    eye = lax.broadcasted_iota(jnp.int32, (1, E), 1)
    for e in range(E):
        end_e = jnp.sum(jnp.where(eye == e, ends, 0.0), axis=1, keepdims=True)
        te += (end_e <= ts).astype(jnp.float32)
    te = jnp.minimum(te, float(E - 1)).astype(jnp.int32)
    act = (ts < total).astype(jnp.int32)
    te_ref[...] = jnp.broadcast_to(te, te_ref.shape)
    act_ref[...] = jnp.broadcast_to(act, act_ref.shape)


def _k1b(cnt, idx, rank):
    return pl.pallas_call(
        _k1b_body,
        grid=(1,),
        in_specs=[
            pl.BlockSpec((8, E), lambda i: (0, 0)),
            pl.BlockSpec((T, TOPK), lambda i: (0, 0)),
            pl.BlockSpec((T, TOPK), lambda i: (0, 0)),
        ],
        out_specs=[
            pl.BlockSpec((T, TOPK), lambda i: (0, 0)),
            pl.BlockSpec((8, 64), lambda i: (0, 0)),
            pl.BlockSpec((8, 64), lambda i: (0, 0)),
        ],
        out_shape=[
            jax.ShapeDtypeStruct((T, TOPK), jnp.int32),
            jax.ShapeDtypeStruct((8, 64), jnp.int32),
            jax.ShapeDtypeStruct((8, 64), jnp.int32),
        ],
        interpret=_INTERP,
    )(cnt, idx, rank)


def _k2(pos_flat, w_flat):
    mesh = plsc.VectorSubcoreMesh(core_axis_name="c", subcore_axis_name="s")

    @functools.partial(
        pl.kernel, mesh=mesh,
        out_type=[
            jax.ShapeDtypeStruct((NP,), jnp.int32),
            jax.ShapeDtypeStruct((NP,), jnp.float32),
        ],
        scratch_types=[
            pltpu.VMEM((NPAIR,), jnp.int32),
            pltpu.VMEM((NPAIR,), jnp.float32),
            pltpu.VMEM((NP,), jnp.int32),
            pltpu.VMEM((NP,), jnp.float32),
        ],
        compiler_params=pltpu.CompilerParams(needs_layout_passes=False),
    )
    def k2(pos_hbm, w_hbm, st_hbm, sw_hbm, pos_v, w_v, st_v, sw_v):
        wid = lax.axis_index("s") * 2 + lax.axis_index("c")

        @pl.when(wid == 0)
        def _():
            pltpu.sync_copy(pos_hbm, pos_v)
            pltpu.sync_copy(w_hbm, w_v)
            iota = lax.iota(jnp.int32, 16)

            def zero_body(j, _):
                st_v[pl.ds(j * 16, 16)] = jnp.zeros((16,), jnp.int32)
                return 0
            lax.fori_loop(0, NP // 16, zero_body, 0)

            def scat_body(j, _):
                b = j * 16
                p_v = pos_v[pl.ds(b, 16)]
                tok = (b + iota) >> 1
                plsc.store_scatter(st_v, [p_v], tok)
                plsc.store_scatter(sw_v, [p_v], w_v[pl.ds(b, 16)])
                return 0
            lax.fori_loop(0, NPAIR // 16, scat_body, 0)

            pltpu.sync_copy(st_v, st_hbm)
            pltpu.sync_copy(sw_v, sw_hbm)

    return k2(pos_flat, w_flat)


def _k3(x, sorted_token):
    mesh = plsc.VectorSubcoreMesh(core_axis_name="c", subcore_axis_name="s")
    b_per_w = NP // NW
    CH = 32

    @functools.partial(
        pl.kernel, mesh=mesh,
        out_type=jax.ShapeDtypeStruct((NP, H), jnp.float32),
        scratch_types=[
            pltpu.VMEM((b_per_w,), jnp.int32),
            pltpu.VMEM((CH, H), jnp.float32),
            pltpu.SemaphoreType.DMA,
        ],
    )
    def k3(x_hbm, st_hbm, xs_hbm, idx_v, rows_v, sem):
        wid = lax.axis_index("s") * 2 + lax.axis_index("c")
        base = wid * b_per_w
        pltpu.sync_copy(st_hbm.at[pl.ds(base, b_per_w)], idx_v)
        for i in range(b_per_w // CH):
            pltpu.async_copy(x_hbm.at[idx_v.at[pl.ds(i * CH, CH)]], rows_v,
                             sem).wait()
            pltpu.sync_copy(rows_v, xs_hbm.at[pl.ds(base + i * CH, CH)])

    return k3(x, sorted_token)


def _k4_body(te_ref, act_ref, x_ref, sw_ref, w1_ref, w3_ref, w2_ref, y_ref):
    i = pl.program_id(0)

    @pl.when(act_ref[i] == 1)
    def _():
        xs = x_ref[...]
        g = _bf16_dot(xs, w1_ref[0], (((1,), (1,)), ((), ())))
        u = _bf16_dot(xs, w3_ref[0], (((1,), (1,)), ((), ())))
        h = (g * jax.nn.sigmoid(g)) * u
        h = h * sw_ref[0, 0, :][:, None]
        y_ref[...] = _bf16_dot(h, w2_ref[0], (((1,), (1,)), ((), ())))


def _k4(x_sorted, sw3, tile_expert, active, w1, w3, w2):
    grid_spec = pltpu.PrefetchScalarGridSpec(
        num_scalar_prefetch=2,
        grid=(NT,),
        in_specs=[
            pl.BlockSpec((RT, H), lambda i, te, act: (i, 0)),
            pl.BlockSpec((1, 1, RT), lambda i, te, act: (i, 0, 0)),
            pl.BlockSpec((1, FF, H), lambda i, te, act: (te[i], 0, 0)),
            pl.BlockSpec((1, FF, H), lambda i, te, act: (te[i], 0, 0)),
            pl.BlockSpec((1, H, FF), lambda i, te, act: (te[i], 0, 0)),
        ],
        out_specs=pl.BlockSpec((RT, H), lambda i, te, act: (i, 0)),
    )
    return pl.pallas_call(
        _k4_body,
        grid_spec=grid_spec,
        out_shape=jax.ShapeDtypeStruct((NP, H), jnp.float32),
        interpret=_INTERP,
    )(tile_expert, active, x_sorted, sw3, w1, w3, w2)


def _k5(shared_flat, y, pos_flat):
    mesh = plsc.VectorSubcoreMesh(core_axis_name="c", subcore_axis_name="s")
    tok_per_w = T // NW
    SUB = 8

    @functools.partial(
        pl.kernel, mesh=mesh,
        out_type=jax.ShapeDtypeStruct((T * H,), jnp.float32),
        scratch_types=[
            pltpu.VMEM((TOPK * tok_per_w,), jnp.int32),
            pltpu.VMEM((TOPK * SUB, H), jnp.float32),
            pltpu.VMEM((SUB * H,), jnp.float32),
            pltpu.VMEM((SUB * H,), jnp.float32),
            pltpu.SemaphoreType.DMA,
        ],
    )
    def k5(sh_hbm, y_hbm, pos_hbm, out_hbm, pos_v, yrows_v, shb_v, outb_v, sem):
        wid = lax.axis_index("s") * 2 + lax.axis_index("c")
        tok_base = wid * tok_per_w
        pltpu.sync_copy(pos_hbm.at[pl.ds(tok_base * TOPK, TOPK * tok_per_w)],
                        pos_v)
        for it in range(tok_per_w // SUB):
            tok0 = tok_base + it * SUB
            pltpu.async_copy(
                y_hbm.at[pos_v.at[pl.ds(it * SUB * TOPK, SUB * TOPK)]],
                yrows_v, sem).wait()
            pltpu.sync_copy(sh_hbm.at[pl.ds(tok0 * H, SUB * H)], shb_v)
            for r in range(SUB):
                def add_body(cc, _):
                    c = cc * 16
                    y0 = yrows_v[2 * r, pl.ds(c, 16)]
                    y1 = yrows_v[2 * r + 1, pl.ds(c, 16)]
                    sh = shb_v[pl.ds(r * H + c, 16)]
                    outb_v[pl.ds(r * H + c, 16)] = y0 + y1 + sh
                    return 0
                lax.fori_loop(0, H // 16, add_body, 0)
            pltpu.sync_copy(outb_v, out_hbm.at[pl.ds(tok0 * H, SUB * H)])

    return k5(shared_flat, y, pos_flat)


def kernel(hidden_states, gate_w, bias, w1, w3, w2, sh_wg, sh_wu, sh_wd):
    x = hidden_states
    shared, wpair, idx, rank, cnt = _k1(x, gate_w, bias, sh_wg, sh_wu, sh_wd)
    pos, te8, act8 = _k1b(cnt, idx, rank)
    pos_flat = pos.reshape(NPAIR)
    st, sw = _k2(pos_flat, wpair.reshape(NPAIR))
    x_sorted = _k3(x, st)
    sw3 = sw.reshape(NT, 1, RT)
    y = _k4(x_sorted, sw3, te8[0], act8[0], w1, w3, w2)
    out_flat = _k5(shared.reshape(T * H), y, pos_flat)
    return out_flat.reshape(T, H)


# speedup vs baseline: 1.6167x; 1.0800x over previous
import functools

import jax
import jax.numpy as jnp
from jax import lax
from jax.experimental import pallas as pl
from jax.experimental.pallas import tpu as pltpu
from jax.experimental.pallas import tpu_sc as plsc

E = 16
TOPK = 2
H = 2048
FF = 1024
SHARED_FF = 1024
T = 2048

NPAIR = T * TOPK
RT = 128
NT = NPAIR // RT + E
NP = NT * RT
NW = 32

_INTERP = False


def _bf16_dot(a, b, dims):
    return lax.dot_general(a.astype(jnp.bfloat16), b.astype(jnp.bfloat16),
                           dims, preferred_element_type=jnp.float32)


def _k1_body(x_ref, gate_w_ref, bias_ref, wg_ref, wu_ref, wd_ref,
             shared_ref, wpair_ref, idx_ref, rank_ref, cnt_ref, x16_ref,
             carry_ref):
    t = pl.program_id(0)
    x = x_ref[...]
    xr = lax.bitcast_convert_type(x.astype(jnp.bfloat16).astype(jnp.float32),
                                  jnp.int32)
    rb = lax.shift_right_logical(xr, 16)
    x16_ref[...] = rb[:, :H // 2] | (rb[:, H // 2:] << 16)
    g = _bf16_dot(x, wg_ref[...], (((1,), (1,)), ((), ())))
    u = _bf16_dot(x, wu_ref[...], (((1,), (1,)), ((), ())))
    h = (g * jax.nn.sigmoid(g)) * u
    shared_ref[...] = _bf16_dot(h, wd_ref[...], (((1,), (1,)), ((), ())))
    logits = lax.dot_general(x, gate_w_ref[...], (((1,), (1,)), ((), ())),
                             preferred_element_type=jnp.float32)
    s = jax.nn.sigmoid(logits)
    sc = s + bias_ref[...]
    iota = lax.broadcasted_iota(jnp.int32, sc.shape, 1)
    m1 = jnp.max(sc, axis=1, keepdims=True)
    i1 = jnp.min(jnp.where(sc == m1, iota, E), axis=1, keepdims=True)
    sc2 = jnp.where(iota == i1, -jnp.inf, sc)
    m2 = jnp.max(sc2, axis=1, keepdims=True)
    i2 = jnp.min(jnp.where(sc2 == m2, iota, E), axis=1, keepdims=True)
    w1v = jnp.sum(jnp.where(iota == i1, s, 0.0), axis=1, keepdims=True)
    w2v = jnp.sum(jnp.where(iota == i2, s, 0.0), axis=1, keepdims=True)
    denom = w1v + w2v
    wpair_ref[...] = jnp.concatenate([w1v / denom, w2v / denom], axis=1)
    idx_ref[...] = jnp.concatenate([i1, i2], axis=1)

    @pl.when(t == 0)
    def _init_carry():
        carry_ref[...] = jnp.zeros_like(carry_ref)

    oh1 = (iota == i1).astype(jnp.float32)
    oh2 = (iota == i2).astype(jnp.float32)
    ohs = oh1 + oh2
    bt = oh1.shape[0]
    r_io = lax.broadcasted_iota(jnp.int32, (bt, bt), 0)
    c_io = lax.broadcasted_iota(jnp.int32, (bt, bt), 1)
    tril = (r_io > c_io).astype(jnp.float32)
    cum = lax.dot_general(tril, ohs, (((1,), (0,)), ((), ())),
                          preferred_element_type=jnp.float32)
    base = cum + carry_ref[...]
    r1 = jnp.sum(oh1 * base, axis=1, keepdims=True)
    r2 = jnp.sum(oh2 * base, axis=1, keepdims=True)
    rank_ref[...] = jnp.concatenate([r1, r2], axis=1).astype(jnp.int32)
    carry_ref[...] += jnp.sum(ohs, axis=0, keepdims=True)
    cnt_ref[...] = jnp.broadcast_to(carry_ref[...], cnt_ref.shape).astype(jnp.int32)


def _k1(x, gate_w, bias, sh_wg, sh_wu, sh_wd):
    BT = 256
    return pl.pallas_call(
        _k1_body,
        grid=(T // BT,),
        in_specs=[
            pl.BlockSpec((BT, H), lambda t: (t, 0)),
            pl.BlockSpec((E, H), lambda t: (0, 0)),
            pl.BlockSpec((1, E), lambda t: (0, 0)),
            pl.BlockSpec((SHARED_FF, H), lambda t: (0, 0)),
            pl.BlockSpec((SHARED_FF, H), lambda t: (0, 0)),
            pl.BlockSpec((H, SHARED_FF), lambda t: (0, 0)),
        ],
        out_specs=[
            pl.BlockSpec((BT, H), lambda t: (t, 0)),
            pl.BlockSpec((BT, TOPK), lambda t: (t, 0)),
            pl.BlockSpec((BT, TOPK), lambda t: (t, 0)),
            pl.BlockSpec((BT, TOPK), lambda t: (t, 0)),
            pl.BlockSpec((8, E), lambda t: (0, 0)),
            pl.BlockSpec((BT, H // 2), lambda t: (t, 0)),
        ],
        out_shape=[
            jax.ShapeDtypeStruct((T, H), jnp.float32),
            jax.ShapeDtypeStruct((T, TOPK), jnp.float32),
            jax.ShapeDtypeStruct((T, TOPK), jnp.int32),
            jax.ShapeDtypeStruct((T, TOPK), jnp.int32),
            jax.ShapeDtypeStruct((8, E), jnp.int32),
            jax.ShapeDtypeStruct((T, H // 2), jnp.int32),
        ],
        scratch_shapes=[pltpu.VMEM((1, E), jnp.float32)],
        interpret=_INTERP,
    )(x, gate_w, bias, sh_wg, sh_wu, sh_wd)


def _take(v, idxv):
    return lax.gather(
        v, idxv[:, None],
        lax.GatherDimensionNumbers(offset_dims=(), collapsed_slice_dims=(0,),
                                   start_index_map=(0,)),
        (1,), mode=lax.GatherScatterMode.PROMISE_IN_BOUNDS)


def _bcast_lane(v, j):
    return _take(v, jnp.full((16,), j, jnp.int32))


def _k1b_body(cnt_ref, idx_ref, rank_ref, pos_ref, te_ref, act_ref):
    cnt = cnt_ref[0:1, :]
    padc = ((cnt + (RT - 1)) >> 7) << 7
    padc_f = padc.astype(jnp.float32)
    r_io = lax.broadcasted_iota(jnp.int32, (E, E), 0)
    c_io = lax.broadcasted_iota(jnp.int32, (E, E), 1)
    supper = (r_io < c_io).astype(jnp.float32)
    po = lax.dot_general(padc_f, supper, (((1,), (0,)), ((), ())),
                         preferred_element_type=jnp.float32)
    ends = po + padc_f
    total = jnp.sum(padc_f, axis=1, keepdims=True)
    iota = lax.broadcasted_iota(jnp.int32, (T, E), 1)
    i1 = idx_ref[:, 0:1]
    i2 = idx_ref[:, 1:2]
    p1 = jnp.sum(jnp.where(iota == i1, po, 0.0), axis=1, keepdims=True)
    p2 = jnp.sum(jnp.where(iota == i2, po, 0.0), axis=1, keepdims=True)
    pos = jnp.concatenate([p1, p2], axis=1).astype(jnp.int32) + rank_ref[...]
    pos_ref[...] = pos
    lane64 = lax.broadcasted_iota(jnp.int32, (1, 64), 1)
    ts = (lane64 * RT).astype(jnp.float32)
    te = jnp.zeros((1, 64), jnp.float32)
    eye = lax.broadcasted_iota(jnp.int32, (1, E), 1)
    for e in range(E):
        end_e = jnp.sum(jnp.where(eye == e, ends, 0.0), axis=1, keepdims=True)
        te += (end_e <= ts).astype(jnp.float32)
    te = jnp.minimum(te, float(E - 1)).astype(jnp.int32)
    act = (ts < total).astype(jnp.int32)
    te_ref[...] = jnp.broadcast_to(te, te_ref.shape)
    act_ref[...] = jnp.broadcast_to(act, act_ref.shape)


def _k1b(cnt, idx, rank):
    return pl.pallas_call(
        _k1b_body,
        grid=(1,),
        in_specs=[
            pl.BlockSpec((8, E), lambda i: (0, 0)),
            pl.BlockSpec((T, TOPK), lambda i: (0, 0)),
            pl.BlockSpec((T, TOPK), lambda i: (0, 0)),
        ],
        out_specs=[
            pl.BlockSpec((T, TOPK), lambda i: (0, 0)),
            pl.BlockSpec((8, 64), lambda i: (0, 0)),
            pl.BlockSpec((8, 64), lambda i: (0, 0)),
        ],
        out_shape=[
            jax.ShapeDtypeStruct((T, TOPK), jnp.int32),
            jax.ShapeDtypeStruct((8, 64), jnp.int32),
            jax.ShapeDtypeStruct((8, 64), jnp.int32),
        ],
        interpret=_INTERP,
    )(cnt, idx, rank)


def _k2(pos_flat, w_flat):
    mesh = plsc.VectorSubcoreMesh(core_axis_name="c", subcore_axis_name="s")

    @functools.partial(
        pl.kernel, mesh=mesh,
        out_type=[
            jax.ShapeDtypeStruct((NP,), jnp.int32),
            jax.ShapeDtypeStruct((NP,), jnp.float32),
        ],
        scratch_types=[
            pltpu.VMEM((NPAIR,), jnp.int32),
            pltpu.VMEM((NPAIR,), jnp.float32),
            pltpu.VMEM((NP,), jnp.int32),
            pltpu.VMEM((NP,), jnp.float32),
        ],
        compiler_params=pltpu.CompilerParams(needs_layout_passes=False),
    )
    def k2(pos_hbm, w_hbm, st_hbm, sw_hbm, pos_v, w_v, st_v, sw_v):
        wid = lax.axis_index("s") * 2 + lax.axis_index("c")

        @pl.when(wid == 0)
        def _():
            pltpu.sync_copy(pos_hbm, pos_v)
            pltpu.sync_copy(w_hbm, w_v)
            iota = lax.iota(jnp.int32, 16)

            def zero_body(j, _):
                st_v[pl.ds(j * 16, 16)] = jnp.zeros((16,), jnp.int32)
                return 0
            lax.fori_loop(0, NP // 16, zero_body, 0)

            def scat_body(j, _):
                b = j * 16
                p_v = pos_v[pl.ds(b, 16)]
                tok = (b + iota) >> 1
                plsc.store_scatter(st_v, [p_v], tok)
                plsc.store_scatter(sw_v, [p_v], w_v[pl.ds(b, 16)])
                return 0
            lax.fori_loop(0, NPAIR // 16, scat_body, 0)

            pltpu.sync_copy(st_v, st_hbm)
            pltpu.sync_copy(sw_v, sw_hbm)

    return k2(pos_flat, w_flat)


def _k3(x16, sorted_token):
    mesh = plsc.VectorSubcoreMesh(core_axis_name="c", subcore_axis_name="s")
    b_per_w = NP // NW
    CH = 48
    NCH = b_per_w // CH

    @functools.partial(
        pl.kernel, mesh=mesh,
        out_type=jax.ShapeDtypeStruct((NP, H // 2), jnp.int32),
        scratch_types=[
            pltpu.VMEM((b_per_w,), jnp.int32),
            pltpu.VMEM((CH, H // 2), jnp.int32),
            pltpu.VMEM((CH, H // 2), jnp.int32),
            pltpu.SemaphoreType.DMA,
            pltpu.SemaphoreType.DMA,
        ],
    )
    def k3(x_hbm, st_hbm, xs_hbm, idx_v, rows0_v, rows1_v, sem0, sem1):
        wid = lax.axis_index("s") * 2 + lax.axis_index("c")
        base = wid * b_per_w
        pltpu.sync_copy(st_hbm.at[pl.ds(base, b_per_w)], idx_v)
        bufs = (rows0_v, rows1_v)
        sems = (sem0, sem1)
        copies = [None] * NCH
        copies[0] = pltpu.async_copy(
            x_hbm.at[idx_v.at[pl.ds(0, CH)]], bufs[0], sems[0])
        for i in range(NCH):
            copies[i].wait()
            if i + 1 < NCH:
                copies[i + 1] = pltpu.async_copy(
                    x_hbm.at[idx_v.at[pl.ds((i + 1) * CH, CH)]],
                    bufs[(i + 1) % 2], sems[(i + 1) % 2])
            pltpu.sync_copy(bufs[i % 2], xs_hbm.at[pl.ds(base + i * CH, CH)])

    return k3(x16, sorted_token)


def _k4_body(te_ref, act_ref, x_ref, sw_ref, w1_ref, w3_ref, w2_ref, y_ref):
    i = pl.program_id(0)

    @pl.when(act_ref[i] == 1)
    def _():
        xp = x_ref[...]
        lo = lax.bitcast_convert_type(xp << 16, jnp.float32)
        hi = lax.bitcast_convert_type(xp & jnp.int32(-65536), jnp.float32)
        xs = jnp.concatenate([lo, hi], axis=1)
        g = _bf16_dot(xs, w1_ref[0], (((1,), (1,)), ((), ())))
        u = _bf16_dot(xs, w3_ref[0], (((1,), (1,)), ((), ())))
        h = (g * jax.nn.sigmoid(g)) * u
        h = h * sw_ref[0, 0, :][:, None]
        y_ref[...] = _bf16_dot(h, w2_ref[0], (((1,), (1,)), ((), ())))


def _k4(x_sorted, sw3, tile_expert, active, w1, w3, w2):
    grid_spec = pltpu.PrefetchScalarGridSpec(
        num_scalar_prefetch=2,
        grid=(NT,),
        in_specs=[
            pl.BlockSpec((RT, H // 2), lambda i, te, act: (i, 0)),
            pl.BlockSpec((1, 1, RT), lambda i, te, act: (i, 0, 0)),
            pl.BlockSpec((1, FF, H), lambda i, te, act: (te[i], 0, 0)),
            pl.BlockSpec((1, FF, H), lambda i, te, act: (te[i], 0, 0)),
            pl.BlockSpec((1, H, FF), lambda i, te, act: (te[i], 0, 0)),
        ],
        out_specs=pl.BlockSpec((RT, H), lambda i, te, act: (i, 0)),
    )
    return pl.pallas_call(
        _k4_body,
        grid_spec=grid_spec,
        out_shape=jax.ShapeDtypeStruct((NP, H), jnp.float32),
        interpret=_INTERP,
    )(tile_expert, active, x_sorted, sw3, w1, w3, w2)


def _k5(shared_flat, y, pos_flat):
    mesh = plsc.VectorSubcoreMesh(core_axis_name="c", subcore_axis_name="s")
    tok_per_w = T // NW
    SUB = 8

    NIT = tok_per_w // SUB

    @functools.partial(
        pl.kernel, mesh=mesh,
        out_type=jax.ShapeDtypeStruct((T * H,), jnp.float32),
        scratch_types=[
            pltpu.VMEM((TOPK * tok_per_w,), jnp.int32),
            pltpu.VMEM((TOPK * SUB, H), jnp.float32),
            pltpu.VMEM((TOPK * SUB, H), jnp.float32),
            pltpu.VMEM((SUB * H,), jnp.float32),
            pltpu.VMEM((SUB * H,), jnp.float32),
            pltpu.SemaphoreType.DMA,
            pltpu.SemaphoreType.DMA,
        ],
    )
    def k5(sh_hbm, y_hbm, pos_hbm, out_hbm, pos_v, yr0_v, yr1_v, shb_v,
           outb_v, sem0, sem1):
        wid = lax.axis_index("s") * 2 + lax.axis_index("c")
        tok_base = wid * tok_per_w
        pltpu.sync_copy(pos_hbm.at[pl.ds(tok_base * TOPK, TOPK * tok_per_w)],
                        pos_v)
        bufs = (yr0_v, yr1_v)
        sems = (sem0, sem1)
        copies = [None] * NIT
        copies[0] = pltpu.async_copy(
            y_hbm.at[pos_v.at[pl.ds(0, SUB * TOPK)]], bufs[0], sems[0])
        for it in range(NIT):
            tok0 = tok_base + it * SUB
            yrows_v = bufs[it % 2]
            copies[it].wait()
            if it + 1 < NIT:
                copies[it + 1] = pltpu.async_copy(
                    y_hbm.at[pos_v.at[pl.ds((it + 1) * SUB * TOPK,
                                            SUB * TOPK)]],
                    bufs[(it + 1) % 2], sems[(it + 1) % 2])
            pltpu.sync_copy(sh_hbm.at[pl.ds(tok0 * H, SUB * H)], shb_v)
            for r in range(SUB):
                def add_body(cc, _):
                    c = cc * 16
                    y0 = yrows_v[2 * r, pl.ds(c, 16)]
                    y1 = yrows_v[2 * r + 1, pl.ds(c, 16)]
                    sh = shb_v[pl.ds(r * H + c, 16)]
                    outb_v[pl.ds(r * H + c, 16)] = y0 + y1 + sh
                    return 0
                lax.fori_loop(0, H // 16, add_body, 0)
            pltpu.sync_copy(outb_v, out_hbm.at[pl.ds(tok0 * H, SUB * H)])

    return k5(shared_flat, y, pos_flat)


def kernel(hidden_states, gate_w, bias, w1, w3, w2, sh_wg, sh_wu, sh_wd):
    x = hidden_states
    shared, wpair, idx, rank, cnt, x16 = _k1(x, gate_w, bias, sh_wg, sh_wu,
                                             sh_wd)
    pos, te8, act8 = _k1b(cnt, idx, rank)
    pos_flat = pos.reshape(NPAIR)
    st, sw = _k2(pos_flat, wpair.reshape(NPAIR))
    x_sorted = _k3(x16, st)
    sw3 = sw.reshape(NT, 1, RT)
    y = _k4(x_sorted, sw3, te8[0], act8[0], w1, w3, w2)
    out_flat = _k5(shared.reshape(T * H), y, pos_flat)
    return out_flat.reshape(T, H)


# speedup vs baseline: 1.6920x; 1.0466x over previous
import functools

import jax
import jax.numpy as jnp
from jax import lax
from jax.experimental import pallas as pl
from jax.experimental.pallas import tpu as pltpu
from jax.experimental.pallas import tpu_sc as plsc

E = 16
TOPK = 2
H = 2048
FF = 1024
SHARED_FF = 1024
T = 2048

NPAIR = T * TOPK
RT = 128
NT = NPAIR // RT + E
NP = NT * RT
NW = 32

_INTERP = False


def _bf16_dot(a, b, dims):
    return lax.dot_general(a.astype(jnp.bfloat16), b.astype(jnp.bfloat16),
                           dims, preferred_element_type=jnp.float32)


def _k1_body(x_ref, gate_w_ref, bias_ref, wg_ref, wu_ref, wd_ref,
             shared_ref, wpair_ref, idx_ref, rank_ref, cnt_ref, x16_ref,
             carry_ref):
    t = pl.program_id(0)
    x = x_ref[...]
    xr = lax.bitcast_convert_type(x.astype(jnp.bfloat16).astype(jnp.float32),
                                  jnp.int32)
    rb = lax.shift_right_logical(xr, 16)
    x16_ref[...] = rb[:, :H // 2] | (rb[:, H // 2:] << 16)
    g = _bf16_dot(x, wg_ref[...], (((1,), (1,)), ((), ())))
    u = _bf16_dot(x, wu_ref[...], (((1,), (1,)), ((), ())))
    h = (g * jax.nn.sigmoid(g)) * u
    shared_ref[...] = _bf16_dot(h, wd_ref[...], (((1,), (1,)), ((), ())))
    logits = lax.dot_general(x, gate_w_ref[...], (((1,), (1,)), ((), ())),
                             preferred_element_type=jnp.float32)
    s = jax.nn.sigmoid(logits)
    sc = s + bias_ref[...]
    iota = lax.broadcasted_iota(jnp.int32, sc.shape, 1)
    m1 = jnp.max(sc, axis=1, keepdims=True)
    i1 = jnp.min(jnp.where(sc == m1, iota, E), axis=1, keepdims=True)
    sc2 = jnp.where(iota == i1, -jnp.inf, sc)
    m2 = jnp.max(sc2, axis=1, keepdims=True)
    i2 = jnp.min(jnp.where(sc2 == m2, iota, E), axis=1, keepdims=True)
    w1v = jnp.sum(jnp.where(iota == i1, s, 0.0), axis=1, keepdims=True)
    w2v = jnp.sum(jnp.where(iota == i2, s, 0.0), axis=1, keepdims=True)
    denom = w1v + w2v
    wpair_ref[...] = jnp.concatenate([w1v / denom, w2v / denom], axis=1)
    idx_ref[...] = jnp.concatenate([i1, i2], axis=1)

    @pl.when(t == 0)
    def _init_carry():
        carry_ref[...] = jnp.zeros_like(carry_ref)

    oh1 = (iota == i1).astype(jnp.float32)
    oh2 = (iota == i2).astype(jnp.float32)
    ohs = oh1 + oh2
    bt = oh1.shape[0]
    r_io = lax.broadcasted_iota(jnp.int32, (bt, bt), 0)
    c_io = lax.broadcasted_iota(jnp.int32, (bt, bt), 1)
    tril = (r_io > c_io).astype(jnp.float32)
    cum = lax.dot_general(tril, ohs, (((1,), (0,)), ((), ())),
                          preferred_element_type=jnp.float32)
    base = cum + carry_ref[...]
    r1 = jnp.sum(oh1 * base, axis=1, keepdims=True)
    r2 = jnp.sum(oh2 * base, axis=1, keepdims=True)
    rank_ref[...] = jnp.concatenate([r1, r2], axis=1).astype(jnp.int32)
    carry_ref[...] += jnp.sum(ohs, axis=0, keepdims=True)
    cnt_ref[...] = jnp.broadcast_to(carry_ref[...], cnt_ref.shape).astype(jnp.int32)


def _k1(x, gate_w, bias, sh_wg, sh_wu, sh_wd):
    BT = 256
    return pl.pallas_call(
        _k1_body,
        grid=(T // BT,),
        in_specs=[
            pl.BlockSpec((BT, H), lambda t: (t, 0)),
            pl.BlockSpec((E, H), lambda t: (0, 0)),
            pl.BlockSpec((1, E), lambda t: (0, 0)),
            pl.BlockSpec((SHARED_FF, H), lambda t: (0, 0)),
            pl.BlockSpec((SHARED_FF, H), lambda t: (0, 0)),
            pl.BlockSpec((H, SHARED_FF), lambda t: (0, 0)),
        ],
        out_specs=[
            pl.BlockSpec((BT, H), lambda t: (t, 0)),
            pl.BlockSpec((BT, TOPK), lambda t: (t, 0)),
            pl.BlockSpec((BT, TOPK), lambda t: (t, 0)),
            pl.BlockSpec((BT, TOPK), lambda t: (t, 0)),
            pl.BlockSpec((8, E), lambda t: (0, 0)),
            pl.BlockSpec((BT, H // 2), lambda t: (t, 0)),
        ],
        out_shape=[
            jax.ShapeDtypeStruct((T, H), jnp.float32),
            jax.ShapeDtypeStruct((T, TOPK), jnp.float32),
            jax.ShapeDtypeStruct((T, TOPK), jnp.int32),
            jax.ShapeDtypeStruct((T, TOPK), jnp.int32),
            jax.ShapeDtypeStruct((8, E), jnp.int32),
            jax.ShapeDtypeStruct((T, H // 2), jnp.int32),
        ],
        scratch_shapes=[pltpu.VMEM((1, E), jnp.float32)],
        interpret=_INTERP,
    )(x, gate_w, bias, sh_wg, sh_wu, sh_wd)


def _take(v, idxv):
    return lax.gather(
        v, idxv[:, None],
        lax.GatherDimensionNumbers(offset_dims=(), collapsed_slice_dims=(0,),
                                   start_index_map=(0,)),
        (1,), mode=lax.GatherScatterMode.PROMISE_IN_BOUNDS)


def _bcast_lane(v, j):
    return _take(v, jnp.full((16,), j, jnp.int32))


def _k1b_body(cnt_ref, idx_ref, rank_ref, pos_ref, te_ref, act_ref):
    cnt = cnt_ref[0:1, :]
    padc = ((cnt + (RT - 1)) >> 7) << 7
    padc_f = padc.astype(jnp.float32)
    r_io = lax.broadcasted_iota(jnp.int32, (E, E), 0)
    c_io = lax.broadcasted_iota(jnp.int32, (E, E), 1)
    supper = (r_io < c_io).astype(jnp.float32)
    po = lax.dot_general(padc_f, supper, (((1,), (0,)), ((), ())),
                         preferred_element_type=jnp.float32)
    ends = po + padc_f
    total = jnp.sum(padc_f, axis=1, keepdims=True)
    iota = lax.broadcasted_iota(jnp.int32, (T, E), 1)
    i1 = idx_ref[:, 0:1]
    i2 = idx_ref[:, 1:2]
    p1 = jnp.sum(jnp.where(iota == i1, po, 0.0), axis=1, keepdims=True)
    p2 = jnp.sum(jnp.where(iota == i2, po, 0.0), axis=1, keepdims=True)
    pos = jnp.concatenate([p1, p2], axis=1).astype(jnp.int32) + rank_ref[...]
    pos_ref[...] = pos
    lane64 = lax.broadcasted_iota(jnp.int32, (1, 64), 1)
    ts = (lane64 * RT).astype(jnp.float32)
    te = jnp.zeros((1, 64), jnp.float32)
    eye = lax.broadcasted_iota(jnp.int32, (1, E), 1)
    for e in range(E):
        end_e = jnp.sum(jnp.where(eye == e, ends, 0.0), axis=1, keepdims=True)
        te += (end_e <= ts).astype(jnp.float32)
    te = jnp.minimum(te, float(E - 1)).astype(jnp.int32)
    act = (ts < total).astype(jnp.int32)
    te_ref[...] = jnp.broadcast_to(te, te_ref.shape)
    act_ref[...] = jnp.broadcast_to(act, act_ref.shape)


def _k1b(cnt, idx, rank):
    return pl.pallas_call(
        _k1b_body,
        grid=(1,),
        in_specs=[
            pl.BlockSpec((8, E), lambda i: (0, 0)),
            pl.BlockSpec((T, TOPK), lambda i: (0, 0)),
            pl.BlockSpec((T, TOPK), lambda i: (0, 0)),
        ],
        out_specs=[
            pl.BlockSpec((T, TOPK), lambda i: (0, 0)),
            pl.BlockSpec((8, 64), lambda i: (0, 0)),
            pl.BlockSpec((8, 64), lambda i: (0, 0)),
        ],
        out_shape=[
            jax.ShapeDtypeStruct((T, TOPK), jnp.int32),
            jax.ShapeDtypeStruct((8, 64), jnp.int32),
            jax.ShapeDtypeStruct((8, 64), jnp.int32),
        ],
        interpret=_INTERP,
    )(cnt, idx, rank)


def _k2(pos_flat, w_flat):
    mesh = plsc.VectorSubcoreMesh(core_axis_name="c", subcore_axis_name="s")

    @functools.partial(
        pl.kernel, mesh=mesh,
        out_type=[
            jax.ShapeDtypeStruct((NP,), jnp.int32),
            jax.ShapeDtypeStruct((NP,), jnp.float32),
        ],
        scratch_types=[
            pltpu.VMEM((NPAIR,), jnp.int32),
            pltpu.VMEM((NPAIR,), jnp.float32),
            pltpu.VMEM((NP,), jnp.int32),
            pltpu.VMEM((NP,), jnp.float32),
        ],
        compiler_params=pltpu.CompilerParams(needs_layout_passes=False),
    )
    def k2(pos_hbm, w_hbm, st_hbm, sw_hbm, pos_v, w_v, st_v, sw_v):
        wid = lax.axis_index("s") * 2 + lax.axis_index("c")

        @pl.when(wid == 0)
        def _():
            pltpu.sync_copy(pos_hbm, pos_v)
            pltpu.sync_copy(w_hbm, w_v)
            iota = lax.iota(jnp.int32, 16)

            def zero_body(j, _):
                st_v[pl.ds(j * 16, 16)] = jnp.zeros((16,), jnp.int32)
                return 0
            lax.fori_loop(0, NP // 16, zero_body, 0)

            def scat_body(j, _):
                b = j * 16
                p_v = pos_v[pl.ds(b, 16)]
                tok = (b + iota) >> 1
                plsc.store_scatter(st_v, [p_v], tok)
                plsc.store_scatter(sw_v, [p_v], w_v[pl.ds(b, 16)])
                return 0
            lax.fori_loop(0, NPAIR // 16, scat_body, 0)

            pltpu.sync_copy(st_v, st_hbm)
            pltpu.sync_copy(sw_v, sw_hbm)

    return k2(pos_flat, w_flat)


def _k3(x16, sorted_token):
    mesh = plsc.VectorSubcoreMesh(core_axis_name="c", subcore_axis_name="s")
    b_per_w = NP // NW
    CH = 24
    NCH = b_per_w // CH
    NBUF = 4

    @functools.partial(
        pl.kernel, mesh=mesh,
        out_type=jax.ShapeDtypeStruct((NP, H // 2), jnp.int32),
        scratch_types=[
            pltpu.VMEM((b_per_w,), jnp.int32),
        ] + [pltpu.VMEM((CH, H // 2), jnp.int32)] * NBUF
          + [pltpu.SemaphoreType.DMA] * NBUF,
    )
    def k3(x_hbm, st_hbm, xs_hbm, idx_v, *bufsems):
        bufs = bufsems[:NBUF]
        sems = bufsems[NBUF:]
        wid = lax.axis_index("s") * 2 + lax.axis_index("c")
        base = wid * b_per_w
        pltpu.sync_copy(st_hbm.at[pl.ds(base, b_per_w)], idx_v)
        copies = [None] * NCH
        for i in range(NBUF - 1):
            copies[i] = pltpu.async_copy(
                x_hbm.at[idx_v.at[pl.ds(i * CH, CH)]], bufs[i % NBUF],
                sems[i % NBUF])
        for i in range(NCH):
            copies[i].wait()
            j = i + NBUF - 1
            if j < NCH:
                copies[j] = pltpu.async_copy(
                    x_hbm.at[idx_v.at[pl.ds(j * CH, CH)]], bufs[j % NBUF],
                    sems[j % NBUF])
            pltpu.sync_copy(bufs[i % NBUF],
                            xs_hbm.at[pl.ds(base + i * CH, CH)])

    return k3(x16, sorted_token)


def _k4_body(te_ref, act_ref, x_ref, sw_ref, w1_ref, w3_ref, w2_ref, y_ref):
    i = pl.program_id(0)

    @pl.when(act_ref[i] == 1)
    def _():
        xp = x_ref[...]
        lo = lax.bitcast_convert_type(xp << 16, jnp.float32)
        hi = lax.bitcast_convert_type(xp & jnp.int32(-65536), jnp.float32)
        xs = jnp.concatenate([lo, hi], axis=1)
        g = _bf16_dot(xs, w1_ref[0], (((1,), (1,)), ((), ())))
        u = _bf16_dot(xs, w3_ref[0], (((1,), (1,)), ((), ())))
        h = (g * jax.nn.sigmoid(g)) * u
        h = h * sw_ref[0, 0, :][:, None]
        y = _bf16_dot(h, w2_ref[0], (((1,), (1,)), ((), ())))
        yr = lax.bitcast_convert_type(
            y.astype(jnp.bfloat16).astype(jnp.float32), jnp.int32)
        rb = lax.shift_right_logical(yr, 16)
        y_ref[...] = rb[:, :H // 2] | (rb[:, H // 2:] << 16)


def _k4(x_sorted, sw3, tile_expert, active, w1, w3, w2):
    grid_spec = pltpu.PrefetchScalarGridSpec(
        num_scalar_prefetch=2,
        grid=(NT,),
        in_specs=[
            pl.BlockSpec((RT, H // 2), lambda i, te, act: (i, 0)),
            pl.BlockSpec((1, 1, RT), lambda i, te, act: (i, 0, 0)),
            pl.BlockSpec((1, FF, H), lambda i, te, act: (te[i], 0, 0)),
            pl.BlockSpec((1, FF, H), lambda i, te, act: (te[i], 0, 0)),
            pl.BlockSpec((1, H, FF), lambda i, te, act: (te[i], 0, 0)),
        ],
        out_specs=pl.BlockSpec((RT, H // 2), lambda i, te, act: (i, 0)),
    )
    return pl.pallas_call(
        _k4_body,
        grid_spec=grid_spec,
        out_shape=jax.ShapeDtypeStruct((NP, H // 2), jnp.int32),
        interpret=_INTERP,
    )(tile_expert, active, x_sorted, sw3, w1, w3, w2)


def _k5(shared_flat, y, pos_flat):
    mesh = plsc.VectorSubcoreMesh(core_axis_name="c", subcore_axis_name="s")
    tok_per_w = T // NW
    SUB = 8

    NIT = tok_per_w // SUB
    NBUF = 4
    HH = H // 2

    @functools.partial(
        pl.kernel, mesh=mesh,
        out_type=jax.ShapeDtypeStruct((T * H,), jnp.float32),
        scratch_types=[
            pltpu.VMEM((TOPK * tok_per_w,), jnp.int32),
            pltpu.VMEM((SUB * H,), jnp.float32),
            pltpu.VMEM((SUB * H,), jnp.float32),
        ] + [pltpu.VMEM((TOPK * SUB, HH), jnp.int32)] * NBUF
          + [pltpu.SemaphoreType.DMA] * NBUF,
        compiler_params=pltpu.CompilerParams(needs_layout_passes=False),
    )
    def k5(sh_hbm, y_hbm, pos_hbm, out_hbm, pos_v, shb_v, outb_v, *bufsems):
        bufs = bufsems[:NBUF]
        sems = bufsems[NBUF:]
        wid = lax.axis_index("s") * 2 + lax.axis_index("c")
        tok_base = wid * tok_per_w
        pltpu.sync_copy(pos_hbm.at[pl.ds(tok_base * TOPK, TOPK * tok_per_w)],
                        pos_v)
        copies = [None] * NIT
        for it in range(NBUF - 1):
            copies[it] = pltpu.async_copy(
                y_hbm.at[pos_v.at[pl.ds(it * SUB * TOPK, SUB * TOPK)]],
                bufs[it % NBUF], sems[it % NBUF])
        mask_hi = jnp.int32(-65536)
        for it in range(NIT):
            tok0 = tok_base + it * SUB
            yrows_v = bufs[it % NBUF]
            copies[it].wait()
            j = it + NBUF - 1
            if j < NIT:
                copies[j] = pltpu.async_copy(
                    y_hbm.at[pos_v.at[pl.ds(j * SUB * TOPK, SUB * TOPK)]],
                    bufs[j % NBUF], sems[j % NBUF])
            pltpu.sync_copy(sh_hbm.at[pl.ds(tok0 * H, SUB * H)], shb_v)
            for r in range(SUB):
                def add_body(cc, _):
                    c = cc * 16
                    y0p = yrows_v[2 * r, pl.ds(c, 16)]
                    y1p = yrows_v[2 * r + 1, pl.ds(c, 16)]
                    lo = (plsc.bitcast(y0p << 16, jnp.float32)
                          + plsc.bitcast(y1p << 16, jnp.float32)
                          + shb_v[pl.ds(r * H + c, 16)])
                    hi = (plsc.bitcast(y0p & mask_hi, jnp.float32)
                          + plsc.bitcast(y1p & mask_hi, jnp.float32)
                          + shb_v[pl.ds(r * H + HH + c, 16)])
                    outb_v[pl.ds(r * H + c, 16)] = lo
                    outb_v[pl.ds(r * H + HH + c, 16)] = hi
                    return 0
                lax.fori_loop(0, HH // 16, add_body, 0)
            pltpu.sync_copy(outb_v, out_hbm.at[pl.ds(tok0 * H, SUB * H)])

    return k5(shared_flat, y, pos_flat)


def kernel(hidden_states, gate_w, bias, w1, w3, w2, sh_wg, sh_wu, sh_wd):
    x = hidden_states
    shared, wpair, idx, rank, cnt, x16 = _k1(x, gate_w, bias, sh_wg, sh_wu,
                                             sh_wd)
    pos, te8, act8 = _k1b(cnt, idx, rank)
    pos_flat = pos.reshape(NPAIR)
    st, sw = _k2(pos_flat, wpair.reshape(NPAIR))
    x_sorted = _k3(x16, st)
    sw3 = sw.reshape(NT, 1, RT)
    y = _k4(x_sorted, sw3, te8[0], act8[0], w1, w3, w2)
    out_flat = _k5(shared.reshape(T * H), y, pos_flat)
    return out_flat.reshape(T, H)


# speedup vs baseline: 1.7467x; 1.0323x over previous
import functools

import jax
import jax.numpy as jnp
from jax import lax
from jax.experimental import pallas as pl
from jax.experimental.pallas import tpu as pltpu
from jax.experimental.pallas import tpu_sc as plsc

E = 16
TOPK = 2
H = 2048
FF = 1024
SHARED_FF = 1024
T = 2048

NPAIR = T * TOPK
RT = 128
NT = NPAIR // RT + E
NP = NT * RT
NW = 32

_INTERP = False


def _bf16_dot(a, b, dims):
    return lax.dot_general(a.astype(jnp.bfloat16), b.astype(jnp.bfloat16),
                           dims, preferred_element_type=jnp.float32)


def _ksh_body(x_ref, wg_ref, wu_ref, wd_ref, shared_ref):
    x = x_ref[...]
    g = _bf16_dot(x, wg_ref[...], (((1,), (1,)), ((), ())))
    u = _bf16_dot(x, wu_ref[...], (((1,), (1,)), ((), ())))
    h = (g * jax.nn.sigmoid(g)) * u
    shared_ref[...] = _bf16_dot(h, wd_ref[...], (((1,), (1,)), ((), ())))


def _ksh(x, sh_wg, sh_wu, sh_wd):
    BT = 256
    return pl.pallas_call(
        _ksh_body,
        grid=(T // BT,),
        in_specs=[
            pl.BlockSpec((BT, H), lambda t: (t, 0)),
            pl.BlockSpec((SHARED_FF, H), lambda t: (0, 0)),
            pl.BlockSpec((SHARED_FF, H), lambda t: (0, 0)),
            pl.BlockSpec((H, SHARED_FF), lambda t: (0, 0)),
        ],
        out_specs=pl.BlockSpec((BT, H), lambda t: (t, 0)),
        out_shape=jax.ShapeDtypeStruct((T, H), jnp.float32),
        interpret=_INTERP,
    )(x, sh_wg, sh_wu, sh_wd)


def _k1_body(x_ref, gate_w_ref, bias_ref,
             wpair_ref, idx_ref, rank_ref, cnt_ref, x16_ref,
             carry_ref):
    t = pl.program_id(0)
    x = x_ref[...]
    xr = lax.bitcast_convert_type(x.astype(jnp.bfloat16).astype(jnp.float32),
                                  jnp.int32)
    rb = lax.shift_right_logical(xr, 16)
    x16_ref[...] = rb[:, :H // 2] | (rb[:, H // 2:] << 16)
    logits = lax.dot_general(x, gate_w_ref[...], (((1,), (1,)), ((), ())),
                             preferred_element_type=jnp.float32)
    s = jax.nn.sigmoid(logits)
    sc = s + bias_ref[...]
    iota = lax.broadcasted_iota(jnp.int32, sc.shape, 1)
    m1 = jnp.max(sc, axis=1, keepdims=True)
    i1 = jnp.min(jnp.where(sc == m1, iota, E), axis=1, keepdims=True)
    sc2 = jnp.where(iota == i1, -jnp.inf, sc)
    m2 = jnp.max(sc2, axis=1, keepdims=True)
    i2 = jnp.min(jnp.where(sc2 == m2, iota, E), axis=1, keepdims=True)
    w1v = jnp.sum(jnp.where(iota == i1, s, 0.0), axis=1, keepdims=True)
    w2v = jnp.sum(jnp.where(iota == i2, s, 0.0), axis=1, keepdims=True)
    denom = w1v + w2v
    wpair_ref[...] = jnp.concatenate([w1v / denom, w2v / denom], axis=1)
    idx_ref[...] = jnp.concatenate([i1, i2], axis=1)

    @pl.when(t == 0)
    def _init_carry():
        carry_ref[...] = jnp.zeros_like(carry_ref)

    oh1 = (iota == i1).astype(jnp.float32)
    oh2 = (iota == i2).astype(jnp.float32)
    ohs = oh1 + oh2
    bt = oh1.shape[0]
    r_io = lax.broadcasted_iota(jnp.int32, (bt, bt), 0)
    c_io = lax.broadcasted_iota(jnp.int32, (bt, bt), 1)
    tril = (r_io > c_io).astype(jnp.float32)
    cum = lax.dot_general(tril, ohs, (((1,), (0,)), ((), ())),
                          preferred_element_type=jnp.float32)
    base = cum + carry_ref[...]
    r1 = jnp.sum(oh1 * base, axis=1, keepdims=True)
    r2 = jnp.sum(oh2 * base, axis=1, keepdims=True)
    rank_ref[...] = jnp.concatenate([r1, r2], axis=1).astype(jnp.int32)
    carry_ref[...] += jnp.sum(ohs, axis=0, keepdims=True)
    cnt_ref[...] = jnp.broadcast_to(carry_ref[...], cnt_ref.shape).astype(jnp.int32)


def _k1(x, gate_w, bias):
    BT = 256
    return pl.pallas_call(
        _k1_body,
        grid=(T // BT,),
        in_specs=[
            pl.BlockSpec((BT, H), lambda t: (t, 0)),
            pl.BlockSpec((E, H), lambda t: (0, 0)),
            pl.BlockSpec((1, E), lambda t: (0, 0)),
        ],
        out_specs=[
            pl.BlockSpec((BT, TOPK), lambda t: (t, 0)),
            pl.BlockSpec((BT, TOPK), lambda t: (t, 0)),
            pl.BlockSpec((BT, TOPK), lambda t: (t, 0)),
            pl.BlockSpec((8, E), lambda t: (0, 0)),
            pl.BlockSpec((BT, H // 2), lambda t: (t, 0)),
        ],
        out_shape=[
            jax.ShapeDtypeStruct((T, TOPK), jnp.float32),
            jax.ShapeDtypeStruct((T, TOPK), jnp.int32),
            jax.ShapeDtypeStruct((T, TOPK), jnp.int32),
            jax.ShapeDtypeStruct((8, E), jnp.int32),
            jax.ShapeDtypeStruct((T, H // 2), jnp.int32),
        ],
        scratch_shapes=[pltpu.VMEM((1, E), jnp.float32)],
        interpret=_INTERP,
    )(x, gate_w, bias)


def _take(v, idxv):
    return lax.gather(
        v, idxv[:, None],
        lax.GatherDimensionNumbers(offset_dims=(), collapsed_slice_dims=(0,),
                                   start_index_map=(0,)),
        (1,), mode=lax.GatherScatterMode.PROMISE_IN_BOUNDS)


def _bcast_lane(v, j):
    return _take(v, jnp.full((16,), j, jnp.int32))


def _k1b_body(cnt_ref, idx_ref, rank_ref, pos_ref, te_ref, act_ref):
    cnt = cnt_ref[0:1, :]
    padc = ((cnt + (RT - 1)) >> 7) << 7
    padc_f = padc.astype(jnp.float32)
    r_io = lax.broadcasted_iota(jnp.int32, (E, E), 0)
    c_io = lax.broadcasted_iota(jnp.int32, (E, E), 1)
    supper = (r_io < c_io).astype(jnp.float32)
    po = lax.dot_general(padc_f, supper, (((1,), (0,)), ((), ())),
                         preferred_element_type=jnp.float32)
    ends = po + padc_f
    total = jnp.sum(padc_f, axis=1, keepdims=True)
    iota = lax.broadcasted_iota(jnp.int32, (T, E), 1)
    i1 = idx_ref[:, 0:1]
    i2 = idx_ref[:, 1:2]
    p1 = jnp.sum(jnp.where(iota == i1, po, 0.0), axis=1, keepdims=True)
    p2 = jnp.sum(jnp.where(iota == i2, po, 0.0), axis=1, keepdims=True)
    pos = jnp.concatenate([p1, p2], axis=1).astype(jnp.int32) + rank_ref[...]
    pos_ref[...] = pos
    lane64 = lax.broadcasted_iota(jnp.int32, (1, 64), 1)
    ts = (lane64 * RT).astype(jnp.float32)
    te = jnp.zeros((1, 64), jnp.float32)
    eye = lax.broadcasted_iota(jnp.int32, (1, E), 1)
    for e in range(E):
        end_e = jnp.sum(jnp.where(eye == e, ends, 0.0), axis=1, keepdims=True)
        te += (end_e <= ts).astype(jnp.float32)
    te = jnp.minimum(te, float(E - 1)).astype(jnp.int32)
    act = (ts < total).astype(jnp.int32)
    te_ref[...] = jnp.broadcast_to(te, te_ref.shape)
    act_ref[...] = jnp.broadcast_to(act, act_ref.shape)


def _k1b(cnt, idx, rank):
    return pl.pallas_call(
        _k1b_body,
        grid=(1,),
        in_specs=[
            pl.BlockSpec((8, E), lambda i: (0, 0)),
            pl.BlockSpec((T, TOPK), lambda i: (0, 0)),
            pl.BlockSpec((T, TOPK), lambda i: (0, 0)),
        ],
        out_specs=[
            pl.BlockSpec((T, TOPK), lambda i: (0, 0)),
            pl.BlockSpec((8, 64), lambda i: (0, 0)),
            pl.BlockSpec((8, 64), lambda i: (0, 0)),
        ],
        out_shape=[
            jax.ShapeDtypeStruct((T, TOPK), jnp.int32),
            jax.ShapeDtypeStruct((8, 64), jnp.int32),
            jax.ShapeDtypeStruct((8, 64), jnp.int32),
        ],
        interpret=_INTERP,
    )(cnt, idx, rank)


def _k2(pos_flat, w_flat):
    mesh = plsc.VectorSubcoreMesh(core_axis_name="c", subcore_axis_name="s")

    @functools.partial(
        pl.kernel, mesh=mesh,
        out_type=[
            jax.ShapeDtypeStruct((NP,), jnp.int32),
            jax.ShapeDtypeStruct((NP,), jnp.float32),
        ],
        scratch_types=[
            pltpu.VMEM((NPAIR,), jnp.int32),
            pltpu.VMEM((NPAIR,), jnp.float32),
            pltpu.VMEM((NP,), jnp.int32),
            pltpu.VMEM((NP,), jnp.float32),
        ],
        compiler_params=pltpu.CompilerParams(needs_layout_passes=False),
    )
    def k2(pos_hbm, w_hbm, st_hbm, sw_hbm, pos_v, w_v, st_v, sw_v):
        wid = lax.axis_index("s") * 2 + lax.axis_index("c")

        @pl.when(wid == 0)
        def _():
            pltpu.sync_copy(pos_hbm, pos_v)
            pltpu.sync_copy(w_hbm, w_v)
            iota = lax.iota(jnp.int32, 16)

            def zero_body(j, _):
                st_v[pl.ds(j * 16, 16)] = jnp.zeros((16,), jnp.int32)
                return 0
            lax.fori_loop(0, NP // 16, zero_body, 0)

            def scat_body(j, _):
                b = j * 16
                p_v = pos_v[pl.ds(b, 16)]
                tok = (b + iota) >> 1
                plsc.store_scatter(st_v, [p_v], tok)
                plsc.store_scatter(sw_v, [p_v], w_v[pl.ds(b, 16)])
                return 0
            lax.fori_loop(0, NPAIR // 16, scat_body, 0)

            pltpu.sync_copy(st_v, st_hbm)
            pltpu.sync_copy(sw_v, sw_hbm)

    return k2(pos_flat, w_flat)


def _k3(x16, sorted_token):
    mesh = plsc.VectorSubcoreMesh(core_axis_name="c", subcore_axis_name="s")
    b_per_w = NP // NW
    CH = 16
    NCH = b_per_w // CH
    NBUF = 6

    @functools.partial(
        pl.kernel, mesh=mesh,
        out_type=jax.ShapeDtypeStruct((NP, H // 2), jnp.int32),
        scratch_types=[
            pltpu.VMEM((b_per_w,), jnp.int32),
        ] + [pltpu.VMEM((CH, H // 2), jnp.int32)] * NBUF
          + [pltpu.SemaphoreType.DMA] * NBUF,
    )
    def k3(x_hbm, st_hbm, xs_hbm, idx_v, *bufsems):
        bufs = bufsems[:NBUF]
        sems = bufsems[NBUF:]
        wid = lax.axis_index("s") * 2 + lax.axis_index("c")
        base = wid * b_per_w
        pltpu.sync_copy(st_hbm.at[pl.ds(base, b_per_w)], idx_v)
        copies = [None] * NCH
        for i in range(NBUF - 1):
            copies[i] = pltpu.async_copy(
                x_hbm.at[idx_v.at[pl.ds(i * CH, CH)]], bufs[i % NBUF],
                sems[i % NBUF])
        for i in range(NCH):
            copies[i].wait()
            j = i + NBUF - 1
            if j < NCH:
                copies[j] = pltpu.async_copy(
                    x_hbm.at[idx_v.at[pl.ds(j * CH, CH)]], bufs[j % NBUF],
                    sems[j % NBUF])
            pltpu.sync_copy(bufs[i % NBUF],
                            xs_hbm.at[pl.ds(base + i * CH, CH)])

    return k3(x16, sorted_token)


def _k4_body(te_ref, act_ref, x_ref, sw_ref, w1_ref, w3_ref, w2_ref, y_ref):
    i = pl.program_id(0)

    @pl.when(act_ref[i] == 1)
    def _():
        xp = x_ref[...]
        lo = lax.bitcast_convert_type(xp << 16, jnp.float32)
        hi = lax.bitcast_convert_type(xp & jnp.int32(-65536), jnp.float32)
        xs = jnp.concatenate([lo, hi], axis=1)
        g = _bf16_dot(xs, w1_ref[0], (((1,), (1,)), ((), ())))
        u = _bf16_dot(xs, w3_ref[0], (((1,), (1,)), ((), ())))
        h = (g * jax.nn.sigmoid(g)) * u
        h = h * sw_ref[0, 0, :][:, None]
        y = _bf16_dot(h, w2_ref[0], (((1,), (1,)), ((), ())))
        yr = lax.bitcast_convert_type(
            y.astype(jnp.bfloat16).astype(jnp.float32), jnp.int32)
        rb = lax.shift_right_logical(yr, 16)
        y_ref[...] = rb[:, :H // 2] | (rb[:, H // 2:] << 16)


def _k4(x_sorted, sw3, tile_expert, active, w1, w3, w2):
    grid_spec = pltpu.PrefetchScalarGridSpec(
        num_scalar_prefetch=2,
        grid=(NT,),
        in_specs=[
            pl.BlockSpec((RT, H // 2), lambda i, te, act: (i, 0)),
            pl.BlockSpec((1, 1, RT), lambda i, te, act: (i, 0, 0)),
            pl.BlockSpec((1, FF, H), lambda i, te, act: (te[i], 0, 0)),
            pl.BlockSpec((1, FF, H), lambda i, te, act: (te[i], 0, 0)),
            pl.BlockSpec((1, H, FF), lambda i, te, act: (te[i], 0, 0)),
        ],
        out_specs=pl.BlockSpec((RT, H // 2), lambda i, te, act: (i, 0)),
    )
    return pl.pallas_call(
        _k4_body,
        grid_spec=grid_spec,
        out_shape=jax.ShapeDtypeStruct((NP, H // 2), jnp.int32),
        interpret=_INTERP,
    )(tile_expert, active, x_sorted, sw3, w1, w3, w2)


def _k5(shared_flat, y, pos_flat):
    mesh = plsc.VectorSubcoreMesh(core_axis_name="c", subcore_axis_name="s")
    tok_per_w = T // NW
    SUB = 8

    NIT = tok_per_w // SUB
    NBUF = 4
    HH = H // 2

    @functools.partial(
        pl.kernel, mesh=mesh,
        out_type=jax.ShapeDtypeStruct((T * H,), jnp.float32),
        scratch_types=[
            pltpu.VMEM((TOPK * tok_per_w,), jnp.int32),
            pltpu.VMEM((SUB * H,), jnp.float32),
            pltpu.VMEM((SUB * H,), jnp.float32),
        ] + [pltpu.VMEM((TOPK * SUB, HH), jnp.int32)] * NBUF
          + [pltpu.SemaphoreType.DMA] * NBUF,
        compiler_params=pltpu.CompilerParams(needs_layout_passes=False),
    )
    def k5(sh_hbm, y_hbm, pos_hbm, out_hbm, pos_v, shb_v, outb_v, *bufsems):
        bufs = bufsems[:NBUF]
        sems = bufsems[NBUF:]
        wid = lax.axis_index("s") * 2 + lax.axis_index("c")
        tok_base = wid * tok_per_w
        pltpu.sync_copy(pos_hbm.at[pl.ds(tok_base * TOPK, TOPK * tok_per_w)],
                        pos_v)
        copies = [None] * NIT
        for it in range(NBUF - 1):
            copies[it] = pltpu.async_copy(
                y_hbm.at[pos_v.at[pl.ds(it * SUB * TOPK, SUB * TOPK)]],
                bufs[it % NBUF], sems[it % NBUF])
        mask_hi = jnp.int32(-65536)
        for it in range(NIT):
            tok0 = tok_base + it * SUB
            yrows_v = bufs[it % NBUF]
            copies[it].wait()
            j = it + NBUF - 1
            if j < NIT:
                copies[j] = pltpu.async_copy(
                    y_hbm.at[pos_v.at[pl.ds(j * SUB * TOPK, SUB * TOPK)]],
                    bufs[j % NBUF], sems[j % NBUF])
            pltpu.sync_copy(sh_hbm.at[pl.ds(tok0 * H, SUB * H)], shb_v)
            for r in range(SUB):
                def add_body(cc, _):
                    c = cc * 16
                    y0p = yrows_v[2 * r, pl.ds(c, 16)]
                    y1p = yrows_v[2 * r + 1, pl.ds(c, 16)]
                    lo = (plsc.bitcast(y0p << 16, jnp.float32)
                          + plsc.bitcast(y1p << 16, jnp.float32)
                          + shb_v[pl.ds(r * H + c, 16)])
                    hi = (plsc.bitcast(y0p & mask_hi, jnp.float32)
                          + plsc.bitcast(y1p & mask_hi, jnp.float32)
                          + shb_v[pl.ds(r * H + HH + c, 16)])
                    outb_v[pl.ds(r * H + c, 16)] = lo
                    outb_v[pl.ds(r * H + HH + c, 16)] = hi
                    return 0
                lax.fori_loop(0, HH // 16, add_body, 0)
            pltpu.sync_copy(outb_v, out_hbm.at[pl.ds(tok0 * H, SUB * H)])

    return k5(shared_flat, y, pos_flat)


def kernel(hidden_states, gate_w, bias, w1, w3, w2, sh_wg, sh_wu, sh_wd):
    x = hidden_states
    wpair, idx, rank, cnt, x16 = _k1(x, gate_w, bias)
    pos, te8, act8 = _k1b(cnt, idx, rank)
    pos_flat = pos.reshape(NPAIR)
    st, sw = _k2(pos_flat, wpair.reshape(NPAIR))
    x_sorted = _k3(x16, st)
    shared = _ksh(x, sh_wg, sh_wu, sh_wd)
    sw3 = sw.reshape(NT, 1, RT)
    y = _k4(x_sorted, sw3, te8[0], act8[0], w1, w3, w2)
    out_flat = _k5(shared.reshape(T * H), y, pos_flat)
    return out_flat.reshape(T, H)


# speedup vs baseline: 1.8793x; 1.0759x over previous
import functools

import jax
import jax.numpy as jnp
from jax import lax
from jax.experimental import pallas as pl
from jax.experimental.pallas import tpu as pltpu
from jax.experimental.pallas import tpu_sc as plsc

E = 16
TOPK = 2
H = 2048
FF = 1024
SHARED_FF = 1024
T = 2048

NPAIR = T * TOPK
RT = 128
NT = NPAIR // RT + E
NP = NT * RT
NW = 32

_INTERP = False


def _bf16_dot(a, b, dims):
    return lax.dot_general(a.astype(jnp.bfloat16), b.astype(jnp.bfloat16),
                           dims, preferred_element_type=jnp.float32)


def _ksh_body(x_ref, wg_ref, wu_ref, wd_ref, shared_ref):
    x = x_ref[...]
    g = _bf16_dot(x, wg_ref[...], (((1,), (1,)), ((), ())))
    u = _bf16_dot(x, wu_ref[...], (((1,), (1,)), ((), ())))
    h = (g * jax.nn.sigmoid(g)) * u
    shared_ref[...] = _bf16_dot(h, wd_ref[...], (((1,), (1,)), ((), ())))


def _ksh(x, sh_wg, sh_wu, sh_wd):
    BT = 256
    return pl.pallas_call(
        _ksh_body,
        grid=(T // BT,),
        in_specs=[
            pl.BlockSpec((BT, H), lambda t: (t, 0)),
            pl.BlockSpec((SHARED_FF, H), lambda t: (0, 0)),
            pl.BlockSpec((SHARED_FF, H), lambda t: (0, 0)),
            pl.BlockSpec((H, SHARED_FF), lambda t: (0, 0)),
        ],
        out_specs=pl.BlockSpec((BT, H), lambda t: (t, 0)),
        out_shape=jax.ShapeDtypeStruct((T, H), jnp.float32),
        interpret=_INTERP,
    )(x, sh_wg, sh_wu, sh_wd)


def _k1_body(x_ref, gate_w_ref, bias_ref,
             wpair_ref, idx_ref, rank_ref, cnt_ref, x16_ref,
             carry_ref):
    t = pl.program_id(0)
    x = x_ref[...]
    xr = lax.bitcast_convert_type(x.astype(jnp.bfloat16).astype(jnp.float32),
                                  jnp.int32)
    rb = lax.shift_right_logical(xr, 16)
    x16_ref[...] = rb[:, :H // 2] | (rb[:, H // 2:] << 16)
    logits = lax.dot_general(x, gate_w_ref[...], (((1,), (1,)), ((), ())),
                             preferred_element_type=jnp.float32)
    s = jax.nn.sigmoid(logits)
    sc = s + bias_ref[...]
    iota = lax.broadcasted_iota(jnp.int32, sc.shape, 1)
    m1 = jnp.max(sc, axis=1, keepdims=True)
    i1 = jnp.min(jnp.where(sc == m1, iota, E), axis=1, keepdims=True)
    sc2 = jnp.where(iota == i1, -jnp.inf, sc)
    m2 = jnp.max(sc2, axis=1, keepdims=True)
    i2 = jnp.min(jnp.where(sc2 == m2, iota, E), axis=1, keepdims=True)
    w1v = jnp.sum(jnp.where(iota == i1, s, 0.0), axis=1, keepdims=True)
    w2v = jnp.sum(jnp.where(iota == i2, s, 0.0), axis=1, keepdims=True)
    denom = w1v + w2v
    wpair_ref[...] = jnp.concatenate([w1v / denom, w2v / denom], axis=1)
    idx_ref[...] = jnp.concatenate([i1, i2], axis=1)

    @pl.when(t == 0)
    def _init_carry():
        carry_ref[...] = jnp.zeros_like(carry_ref)

    oh1 = (iota == i1).astype(jnp.float32)
    oh2 = (iota == i2).astype(jnp.float32)
    ohs = oh1 + oh2
    bt = oh1.shape[0]
    r_io = lax.broadcasted_iota(jnp.int32, (bt, bt), 0)
    c_io = lax.broadcasted_iota(jnp.int32, (bt, bt), 1)
    tril = (r_io > c_io).astype(jnp.float32)
    cum = lax.dot_general(tril, ohs, (((1,), (0,)), ((), ())),
                          preferred_element_type=jnp.float32)
    base = cum + carry_ref[...]
    r1 = jnp.sum(oh1 * base, axis=1, keepdims=True)
    r2 = jnp.sum(oh2 * base, axis=1, keepdims=True)
    rank_ref[...] = jnp.concatenate([r1, r2], axis=1).astype(jnp.int32)
    carry_ref[...] += jnp.sum(ohs, axis=0, keepdims=True)
    cnt_ref[...] = jnp.broadcast_to(carry_ref[...], cnt_ref.shape).astype(jnp.int32)


def _k1(x, gate_w, bias):
    BT = 256
    return pl.pallas_call(
        _k1_body,
        grid=(T // BT,),
        in_specs=[
            pl.BlockSpec((BT, H), lambda t: (t, 0)),
            pl.BlockSpec((E, H), lambda t: (0, 0)),
            pl.BlockSpec((1, E), lambda t: (0, 0)),
        ],
        out_specs=[
            pl.BlockSpec((BT, TOPK), lambda t: (t, 0)),
            pl.BlockSpec((BT, TOPK), lambda t: (t, 0)),
            pl.BlockSpec((BT, TOPK), lambda t: (t, 0)),
            pl.BlockSpec((8, E), lambda t: (0, 0)),
            pl.BlockSpec((BT, H // 2), lambda t: (t, 0)),
        ],
        out_shape=[
            jax.ShapeDtypeStruct((T, TOPK), jnp.float32),
            jax.ShapeDtypeStruct((T, TOPK), jnp.int32),
            jax.ShapeDtypeStruct((T, TOPK), jnp.int32),
            jax.ShapeDtypeStruct((8, E), jnp.int32),
            jax.ShapeDtypeStruct((T, H // 2), jnp.int32),
        ],
        scratch_shapes=[pltpu.VMEM((1, E), jnp.float32)],
        interpret=_INTERP,
    )(x, gate_w, bias)


def _take(v, idxv):
    return lax.gather(
        v, idxv[:, None],
        lax.GatherDimensionNumbers(offset_dims=(), collapsed_slice_dims=(0,),
                                   start_index_map=(0,)),
        (1,), mode=lax.GatherScatterMode.PROMISE_IN_BOUNDS)


def _bcast_lane(v, j):
    return _take(v, jnp.full((16,), j, jnp.int32))


def _k1b_body(cnt_ref, idx_ref, rank_ref, pos_ref, te_ref, act_ref):
    cnt = cnt_ref[0:1, :]
    padc = ((cnt + (RT - 1)) >> 7) << 7
    padc_f = padc.astype(jnp.float32)
    r_io = lax.broadcasted_iota(jnp.int32, (E, E), 0)
    c_io = lax.broadcasted_iota(jnp.int32, (E, E), 1)
    supper = (r_io < c_io).astype(jnp.float32)
    po = lax.dot_general(padc_f, supper, (((1,), (0,)), ((), ())),
                         preferred_element_type=jnp.float32)
    ends = po + padc_f
    total = jnp.sum(padc_f, axis=1, keepdims=True)
    iota = lax.broadcasted_iota(jnp.int32, (T, E), 1)
    i1 = idx_ref[:, 0:1]
    i2 = idx_ref[:, 1:2]
    p1 = jnp.sum(jnp.where(iota == i1, po, 0.0), axis=1, keepdims=True)
    p2 = jnp.sum(jnp.where(iota == i2, po, 0.0), axis=1, keepdims=True)
    pos = jnp.concatenate([p1, p2], axis=1).astype(jnp.int32) + rank_ref[...]
    pos_ref[...] = pos
    lane64 = lax.broadcasted_iota(jnp.int32, (1, 64), 1)
    ts = (lane64 * RT).astype(jnp.float32)
    te = jnp.zeros((1, 64), jnp.float32)
    eye = lax.broadcasted_iota(jnp.int32, (1, E), 1)
    for e in range(E):
        end_e = jnp.sum(jnp.where(eye == e, ends, 0.0), axis=1, keepdims=True)
        te += (end_e <= ts).astype(jnp.float32)
    te = jnp.minimum(te, float(E - 1)).astype(jnp.int32)
    act = (ts < total).astype(jnp.int32)
    te_ref[...] = jnp.broadcast_to(te, te_ref.shape)
    act_ref[...] = jnp.broadcast_to(act, act_ref.shape)


def _k1b(cnt, idx, rank):
    return pl.pallas_call(
        _k1b_body,
        grid=(1,),
        in_specs=[
            pl.BlockSpec((8, E), lambda i: (0, 0)),
            pl.BlockSpec((T, TOPK), lambda i: (0, 0)),
            pl.BlockSpec((T, TOPK), lambda i: (0, 0)),
        ],
        out_specs=[
            pl.BlockSpec((T, TOPK), lambda i: (0, 0)),
            pl.BlockSpec((8, 64), lambda i: (0, 0)),
            pl.BlockSpec((8, 64), lambda i: (0, 0)),
        ],
        out_shape=[
            jax.ShapeDtypeStruct((T, TOPK), jnp.int32),
            jax.ShapeDtypeStruct((8, 64), jnp.int32),
            jax.ShapeDtypeStruct((8, 64), jnp.int32),
        ],
        interpret=_INTERP,
    )(cnt, idx, rank)


def _k2(pos_flat, w_flat):
    mesh = plsc.VectorSubcoreMesh(core_axis_name="c", subcore_axis_name="s")

    @functools.partial(
        pl.kernel, mesh=mesh,
        out_type=[
            jax.ShapeDtypeStruct((NP,), jnp.int32),
            jax.ShapeDtypeStruct((NP,), jnp.float32),
        ],
        scratch_types=[
            pltpu.VMEM((NPAIR,), jnp.int32),
            pltpu.VMEM((NPAIR,), jnp.float32),
            pltpu.VMEM((NP,), jnp.int32),
            pltpu.VMEM((NP,), jnp.float32),
        ],
        compiler_params=pltpu.CompilerParams(needs_layout_passes=False),
    )
    def k2(pos_hbm, w_hbm, st_hbm, sw_hbm, pos_v, w_v, st_v, sw_v):
        wid = lax.axis_index("s") * 2 + lax.axis_index("c")

        @pl.when(wid == 0)
        def _():
            pltpu.sync_copy(pos_hbm, pos_v)
            pltpu.sync_copy(w_hbm, w_v)
            iota = lax.iota(jnp.int32, 16)

            def zero_body(j, _):
                st_v[pl.ds(j * 16, 16)] = jnp.zeros((16,), jnp.int32)
                return 0
            lax.fori_loop(0, NP // 16, zero_body, 0)

            def scat_body(j, _):
                b = j * 16
                p_v = pos_v[pl.ds(b, 16)]
                tok = (b + iota) >> 1
                plsc.store_scatter(st_v, [p_v], tok)
                plsc.store_scatter(sw_v, [p_v], w_v[pl.ds(b, 16)])
                return 0
            lax.fori_loop(0, NPAIR // 16, scat_body, 0)

            pltpu.sync_copy(st_v, st_hbm)
            pltpu.sync_copy(sw_v, sw_hbm)

    return k2(pos_flat, w_flat)


def _k3(x16, sorted_token):
    mesh = plsc.VectorSubcoreMesh(core_axis_name="c", subcore_axis_name="s")
    b_per_w = NP // NW
    CH = 16
    NCH = b_per_w // CH
    NBUF = 6

    @functools.partial(
        pl.kernel, mesh=mesh,
        out_type=jax.ShapeDtypeStruct((NP, H // 2), jnp.int32),
        scratch_types=[
            pltpu.VMEM((b_per_w,), jnp.int32),
        ] + [pltpu.VMEM((CH, H // 2), jnp.int32)] * NBUF
          + [pltpu.SemaphoreType.DMA] * NBUF,
    )
    def k3(x_hbm, st_hbm, xs_hbm, idx_v, *bufsems):
        bufs = bufsems[:NBUF]
        sems = bufsems[NBUF:]
        wid = lax.axis_index("s") * 2 + lax.axis_index("c")
        base = wid * b_per_w
        pltpu.sync_copy(st_hbm.at[pl.ds(base, b_per_w)], idx_v)
        copies = [None] * NCH
        for i in range(NBUF - 1):
            copies[i] = pltpu.async_copy(
                x_hbm.at[idx_v.at[pl.ds(i * CH, CH)]], bufs[i % NBUF],
                sems[i % NBUF])
        for i in range(NCH):
            copies[i].wait()
            j = i + NBUF - 1
            if j < NCH:
                copies[j] = pltpu.async_copy(
                    x_hbm.at[idx_v.at[pl.ds(j * CH, CH)]], bufs[j % NBUF],
                    sems[j % NBUF])
            pltpu.sync_copy(bufs[i % NBUF],
                            xs_hbm.at[pl.ds(base + i * CH, CH)])

    return k3(x16, sorted_token)


def _k4_body(te_ref, act_ref, x_ref, sw_ref, w1_ref, w3_ref, w2_ref, y_ref):
    i = pl.program_id(0)

    @pl.when(act_ref[i] == 1)
    def _():
        xp = x_ref[...]
        lo = lax.bitcast_convert_type(xp << 16, jnp.float32)
        hi = lax.bitcast_convert_type(xp & jnp.int32(-65536), jnp.float32)
        xs = jnp.concatenate([lo, hi], axis=1)
        g = _bf16_dot(xs, w1_ref[0], (((1,), (1,)), ((), ())))
        u = _bf16_dot(xs, w3_ref[0], (((1,), (1,)), ((), ())))
        h = (g * jax.nn.sigmoid(g)) * u
        h = h * sw_ref[0, 0, :][:, None]
        y = _bf16_dot(h, w2_ref[0], (((1,), (1,)), ((), ())))
        yr = lax.bitcast_convert_type(
            y.astype(jnp.bfloat16).astype(jnp.float32), jnp.int32)
        rb = lax.shift_right_logical(yr, 16)
        y_ref[...] = rb[:, :H // 2] | (rb[:, H // 2:] << 16)


def _k4(x_sorted, sw3, tile_expert, active, w1, w3, w2):
    grid_spec = pltpu.PrefetchScalarGridSpec(
        num_scalar_prefetch=2,
        grid=(NT,),
        in_specs=[
            pl.BlockSpec((RT, H // 2), lambda i, te, act: (i, 0)),
            pl.BlockSpec((1, 1, RT), lambda i, te, act: (i, 0, 0)),
            pl.BlockSpec((1, FF, H), lambda i, te, act: (te[i], 0, 0)),
            pl.BlockSpec((1, FF, H), lambda i, te, act: (te[i], 0, 0)),
            pl.BlockSpec((1, H, FF), lambda i, te, act: (te[i], 0, 0)),
        ],
        out_specs=pl.BlockSpec((RT, H // 2), lambda i, te, act: (i, 0)),
    )
    return pl.pallas_call(
        _k4_body,
        grid_spec=grid_spec,
        out_shape=jax.ShapeDtypeStruct((NP, H // 2), jnp.int32),
        interpret=_INTERP,
    )(tile_expert, active, x_sorted, sw3, w1, w3, w2)


def _k5(shared_flat, y, pos_flat):
    mesh = plsc.VectorSubcoreMesh(core_axis_name="c", subcore_axis_name="s")
    tok_per_w = T // NW
    SUB = 8

    NIT = tok_per_w // SUB
    NBUF = 4
    HH = H // 2

    @functools.partial(
        pl.kernel, mesh=mesh,
        out_type=jax.ShapeDtypeStruct((T, H), jnp.float32),
        scratch_types=[
            pltpu.VMEM((TOPK * tok_per_w,), jnp.int32),
            pltpu.VMEM((SUB, H), jnp.float32),
            pltpu.VMEM((SUB, H), jnp.float32),
        ] + [pltpu.VMEM((TOPK * SUB, HH), jnp.int32)] * NBUF
          + [pltpu.SemaphoreType.DMA] * NBUF,
        compiler_params=pltpu.CompilerParams(needs_layout_passes=False),
    )
    def k5(sh_hbm, y_hbm, pos_hbm, out_hbm, pos_v, shb_v, outb_v, *bufsems):
        bufs = bufsems[:NBUF]
        sems = bufsems[NBUF:]
        wid = lax.axis_index("s") * 2 + lax.axis_index("c")
        tok_base = wid * tok_per_w
        pltpu.sync_copy(pos_hbm.at[pl.ds(tok_base * TOPK, TOPK * tok_per_w)],
                        pos_v)
        copies = [None] * NIT
        for it in range(NBUF - 1):
            copies[it] = pltpu.async_copy(
                y_hbm.at[pos_v.at[pl.ds(it * SUB * TOPK, SUB * TOPK)]],
                bufs[it % NBUF], sems[it % NBUF])
        mask_hi = jnp.int32(-65536)
        for it in range(NIT):
            tok0 = tok_base + it * SUB
            yrows_v = bufs[it % NBUF]
            copies[it].wait()
            j = it + NBUF - 1
            if j < NIT:
                copies[j] = pltpu.async_copy(
                    y_hbm.at[pos_v.at[pl.ds(j * SUB * TOPK, SUB * TOPK)]],
                    bufs[j % NBUF], sems[j % NBUF])
            pltpu.sync_copy(sh_hbm.at[pl.ds(tok0, SUB)], shb_v)
            for r in range(SUB):
                def add_body(cc, _):
                    c = cc * 16
                    y0p = yrows_v[2 * r, pl.ds(c, 16)]
                    y1p = yrows_v[2 * r + 1, pl.ds(c, 16)]
                    lo = (plsc.bitcast(y0p << 16, jnp.float32)
                          + plsc.bitcast(y1p << 16, jnp.float32)
                          + shb_v[r, pl.ds(c, 16)])
                    hi = (plsc.bitcast(y0p & mask_hi, jnp.float32)
                          + plsc.bitcast(y1p & mask_hi, jnp.float32)
                          + shb_v[r, pl.ds(HH + c, 16)])
                    outb_v[r, pl.ds(c, 16)] = lo
                    outb_v[r, pl.ds(HH + c, 16)] = hi
                    return 0
                lax.fori_loop(0, HH // 16, add_body, 0)
            pltpu.sync_copy(outb_v, out_hbm.at[pl.ds(tok0, SUB)])

    return k5(shared_flat, y, pos_flat)


def kernel(hidden_states, gate_w, bias, w1, w3, w2, sh_wg, sh_wu, sh_wd):
    x = hidden_states
    wpair, idx, rank, cnt, x16 = _k1(x, gate_w, bias)
    pos, te8, act8 = _k1b(cnt, idx, rank)
    pos_flat = pos.reshape(NPAIR)
    st, sw = _k2(pos_flat, wpair.reshape(NPAIR))
    x_sorted = _k3(x16, st)
    shared = _ksh(x, sh_wg, sh_wu, sh_wd)
    sw3 = sw.reshape(NT, 1, RT)
    y = _k4(x_sorted, sw3, te8[0], act8[0], w1, w3, w2)
    return _k5(shared, y, pos_flat)
